# Initial kernel scaffold; baseline (speedup 1.0000x reference)
import jax
import jax.numpy as jnp
from jax.experimental import pallas as pl


def kernel(z, memory_bank, Wq, bq, Wk, bk, Wv, bv, Wo, bo):
    raise NotImplementedError("write your pallas kernel here")


# speedup vs baseline: 2.1237x; 2.1237x over previous
import functools
import math

import jax
import jax.numpy as jnp
from jax.experimental import pallas as pl
from jax.experimental.pallas import tpu as pltpu

N_HEADS = 2
D = 192
DK = D // N_HEADS
BANK = 1000
BANKP = 1024
TB = 512


def _kv_kernel(mb_ref, wk_ref, bk_ref, wv_ref, bv_ref, k_ref, v_ref):
    mb = mb_ref[...]
    k_ref[...] = (
        jnp.dot(wk_ref[...], mb, preferred_element_type=jnp.float32) + bk_ref[...]
    )
    v_ref[...] = (
        jnp.dot(wv_ref[...], mb, preferred_element_type=jnp.float32) + bv_ref[...]
    )


def _attn_kernel(z_ref, k_ref, v_ref, wq_ref, bq_ref, wo_ref, bo_ref, o_ref):
    zb = z_ref[0]
    q = jnp.dot(wq_ref[...], zb, preferred_element_type=jnp.float32) + bq_ref[...]

    valid = (
        jax.lax.broadcasted_iota(jnp.int32, (BANKP, TB), 0) < BANK
    )

    outs = []
    for h in range(N_HEADS):
        qh = q[h * DK : (h + 1) * DK, :]
        kh = k_ref[h * DK : (h + 1) * DK, :]
        vh = v_ref[h * DK : (h + 1) * DK, :]
        st = jax.lax.dot_general(
            kh, qh, (((0,), (0,)), ((), ())), preferred_element_type=jnp.float32
        )
        m = jnp.max(st[:BANK], axis=0, keepdims=True)
        e = jnp.exp(st - m)
        e = jnp.where(valid, e, 0.0)
        denom = jnp.sum(e, axis=0, keepdims=True)
        oh = jnp.dot(vh, e, preferred_element_type=jnp.float32)
        outs.append(oh * (1.0 / denom))

    cat = jnp.concatenate(outs, axis=0)
    o_ref[0] = (
        jnp.dot(wo_ref[...], cat, preferred_element_type=jnp.float32) + bo_ref[...]
    )


@jax.jit
def kernel(z, memory_bank, Wq, bq, Wk, bk, Wv, bv, Wo, bo):
    b, d, t = z.shape
    scale = 1.0 / math.sqrt(DK)

    mb_pad = jnp.pad(memory_bank, ((0, 0), (0, BANKP - BANK)))
    wq_s = Wq * scale
    bq_s = (bq * scale)[:, None]
    bk2 = bk[:, None]
    bv2 = bv[:, None]
    bo2 = bo[:, None]

    k, v = pl.pallas_call(
        _kv_kernel,
        out_shape=(
            jax.ShapeDtypeStruct((D, BANKP), jnp.float32),
            jax.ShapeDtypeStruct((D, BANKP), jnp.float32),
        ),
    )(mb_pad, Wk, bk2, Wv, bv2)

    nT = t // TB
    out = pl.pallas_call(
        _attn_kernel,
        out_shape=jax.ShapeDtypeStruct((b, d, t), jnp.float32),
        grid=(b, nT),
        in_specs=[
            pl.BlockSpec((1, D, TB), lambda i, j: (i, 0, j)),
            pl.BlockSpec((D, BANKP), lambda i, j: (0, 0)),
            pl.BlockSpec((D, BANKP), lambda i, j: (0, 0)),
            pl.BlockSpec((D, D), lambda i, j: (0, 0)),
            pl.BlockSpec((D, 1), lambda i, j: (0, 0)),
            pl.BlockSpec((D, D), lambda i, j: (0, 0)),
            pl.BlockSpec((D, 1), lambda i, j: (0, 0)),
        ],
        out_specs=pl.BlockSpec((1, D, TB), lambda i, j: (i, 0, j)),
        compiler_params=pltpu.CompilerParams(
            dimension_semantics=("parallel", "arbitrary"),
        ),
    )(z, k, v, wq_s, bq_s, Wo, bo2)
    return out
